# baseline (speedup 1.0000x reference)
import functools

import jax
import jax.numpy as jnp
from jax import lax
from jax.experimental import pallas as pl
from jax.experimental.pallas import tpu as pltpu
from jax.experimental.pallas import tpu_sc as plsc

RES = 64
PAD = 0.1
DIM = 32
NPL = 3
CELLS = RES * RES
QW = 8
GRIDW = NPL * CELLS * QW
K = 256
NB = 1024


def _cells8(x):
    planes = [(0, 1), (0, 2), (1, 2)]
    cs = []
    for (a, b2) in planes:
        p = jnp.stack([x[..., a], x[..., b2]], axis=-1)
        p = p / (1.0 + PAD + 1e-3) + 0.5
        p = jnp.clip(p, 0.0, 1.0 - 1e-6)
        ij = jnp.clip((p * RES).astype(jnp.int32), 0, RES - 1)
        cs.append(ij[..., 0] + RES * ij[..., 1])
    c = jnp.stack(cs, axis=1)
    return (c * 8).reshape(-1)


def _pack(v, n):
    u = jnp.reshape(v, (n // 4, 4, DIM))
    return jnp.concatenate([u[:, g, :] for g in range(4)], axis=-1)


def _unpack(p, w):
    r = p.shape[0]
    m = 128 // w
    u = jnp.stack([p[:, w * g:w * g + w] for g in range(m)], axis=1)
    return jnp.reshape(u, (r * m, w))


def _stem0_body(x_ref, sw, sb, w0, b0, w1, b1, ws, bs, out_ref):
    x = x_ref[0]
    t = jnp.dot(x, sw[...], preferred_element_type=jnp.float32) + sb[...]
    net = jnp.maximum(t, 0.0)
    net = jnp.dot(net, w0[...], preferred_element_type=jnp.float32) + b0[...]
    net = jnp.maximum(net, 0.0)
    net = jnp.dot(net, w1[...], preferred_element_type=jnp.float32) + b1[...]
    sc = jnp.dot(t, ws[...], preferred_element_type=jnp.float32) + bs[...]
    out_ref[...] = _pack(sc + net, NB)


def _stem0_tc(x, params):
    B, N, _ = x.shape
    p = params
    b0 = p['blocks'][0]
    w_args = (p['stem_W'], p['stem_b'].reshape(1, -1),
              b0['fc0_W'], b0['fc0_b'].reshape(1, -1),
              b0['fc1_W'], b0['fc1_b'].reshape(1, -1),
              b0['sc_W'], b0['sc_b'].reshape(1, -1))
    w_specs = [pl.BlockSpec(w.shape, lambda bb, i: (0, 0)) for w in w_args]
    return pl.pallas_call(
        _stem0_body,
        grid=(B, N // NB),
        in_specs=[pl.BlockSpec((1, NB, 3), lambda bb, i: (bb, i, 0))] + w_specs,
        out_specs=pl.BlockSpec((NB * DIM // 128, 128),
                               lambda bb, i: (bb * (N // NB) + i, 0)),
        out_shape=jax.ShapeDtypeStruct((B * N * DIM // 128, 128), jnp.float32),
    )(x, *w_args)


def _round_body(h_ref, p0, p1, p2, p3, w0, b0, w1, b1, ws, bs, out_ref):
    h = _unpack(h_ref[...], DIM)
    x = jnp.concatenate(
        [h] + [_unpack(p[...], QW) for p in (p0, p1, p2, p3)], axis=-1)
    net = jnp.maximum(x, 0.0)
    net = jnp.dot(net, w0[...], preferred_element_type=jnp.float32) + b0[...]
    net = jnp.maximum(net, 0.0)
    net = jnp.dot(net, w1[...], preferred_element_type=jnp.float32) + b1[...]
    sc = jnp.dot(x, ws[...], preferred_element_type=jnp.float32) + bs[...]
    out_ref[...] = _pack(sc + net, NB)


def _round_tc(h, pooled, blk, B, N):
    w_args = (blk['fc0_W'], blk['fc0_b'].reshape(1, -1),
              blk['fc1_W'], blk['fc1_b'].reshape(1, -1),
              blk['sc_W'], blk['sc_b'].reshape(1, -1))
    w_specs = [pl.BlockSpec(w.shape, lambda bb, i: (0, 0)) for w in w_args]
    q_specs = [
        pl.BlockSpec((NB * QW // 128, 128),
                     functools.partial(
                         lambda bb, i, q: ((bb * 4 + q) * (N // NB) + i, 0),
                         q=q))
        for q in range(4)
    ]
    hspec = pl.BlockSpec((NB * DIM // 128, 128),
                         lambda bb, i: (bb * (N // NB) + i, 0))
    return pl.pallas_call(
        _round_body,
        grid=(B, N // NB),
        in_specs=[hspec] + q_specs + w_specs,
        out_specs=hspec,
        out_shape=jax.ShapeDtypeStruct((B * N * DIM // 128, 128), jnp.float32),
    )(h, pooled, pooled, pooled, pooled, *w_args)


def _fc_body(h_ref, w, b, out_ref, outf_ref):
    c = (jnp.dot(_unpack(h_ref[...], DIM), w[...],
                 preferred_element_type=jnp.float32) + b[...])
    out_ref[0] = c
    outf_ref[...] = _pack(c, NB)


def _fc_tc(h, w, b, B, N):
    w_args = (w, b.reshape(1, -1))
    w_specs = [pl.BlockSpec(a.shape, lambda bb, i: (0, 0)) for a in w_args]
    return pl.pallas_call(
        _fc_body,
        grid=(B, N // NB),
        in_specs=[pl.BlockSpec((NB * DIM // 128, 128),
                               lambda bb, i: (bb * (N // NB) + i, 0))]
        + w_specs,
        out_specs=[pl.BlockSpec((1, NB, DIM), lambda bb, i: (bb, i, 0)),
                   pl.BlockSpec((NB * DIM // 128, 128),
                                lambda bb, i: (bb * (N // NB) + i, 0))],
        out_shape=[jax.ShapeDtypeStruct((B, N, DIM), jnp.float32),
                   jax.ShapeDtypeStruct((B * N * DIM // 128, 128),
                                        jnp.float32)],
    )(h, *w_args)


def _divide_body(s0, s1, s2, s3, cnt_ref, out_ref):
    c = jnp.maximum(_unpack(cnt_ref[...], QW), 1.0)
    out_ref[0, 0] = jnp.concatenate(
        [_unpack(r[...], QW) / c for r in (s0, s1, s2, s3)], axis=-1)


_CC = 1024


def _divide_tc(sums, cnt8, B):
    nrow = _CC * QW // 128
    nch = CELLS // _CC
    q_specs = [
        pl.BlockSpec((nrow, 128),
                     functools.partial(
                         lambda bb, p, j, q:
                         (((bb * 4 + q) * NPL + p) * nch + j, 0), q=q))
        for q in range(4)
    ]
    return pl.pallas_call(
        _divide_body,
        grid=(B, NPL, nch),
        in_specs=q_specs + [pl.BlockSpec(
            (nrow, 128), lambda bb, p, j: ((bb * NPL + p) * nch + j, 0))],
        out_specs=pl.BlockSpec((1, 1, _CC, DIM),
                               lambda bb, p, j: (bb, p, j, 0)),
        out_shape=jax.ShapeDtypeStruct((B, NPL, CELLS, DIM), jnp.float32),
    )(sums, sums, sums, sums, cnt8)


def _dg(x, idx):
    return lax.gather(
        x, idx[:, None],
        lax.GatherDimensionNumbers(
            offset_dims=(), collapsed_slice_dims=(0,), start_index_map=(0,)),
        slice_sizes=(1,),
        mode=lax.GatherScatterMode.PROMISE_IN_BOUNDS)


def _io16():
    return lax.iota(jnp.int32, 16)


def _sc_round_fn(B, N):
    mesh = plsc.VectorSubcoreMesh(core_axis_name="c", subcore_axis_name="s")
    nch = N // K

    @functools.partial(
        pl.kernel, mesh=mesh,
        out_type=jax.ShapeDtypeStruct((B * 4 * N * QW,), jnp.float32),
        scratch_types=[
            pltpu.VMEM((GRIDW,), jnp.float32),
            pltpu.VMEM((NPL * K,), jnp.int32), pltpu.VMEM((NPL * K,), jnp.int32),
            pltpu.VMEM((K * DIM,), jnp.float32), pltpu.VMEM((K * DIM,), jnp.float32),
            pltpu.VMEM((K * QW,), jnp.float32), pltpu.VMEM((K * QW,), jnp.float32),
            pltpu.SemaphoreType.DMA, pltpu.SemaphoreType.DMA,
            pltpu.SemaphoreType.DMA, pltpu.SemaphoreType.DMA,
        ],
        compiler_params=pltpu.CompilerParams(needs_layout_passes=False),
    )
    def k(cells8, hflat, pooled, grid_v, ix0, ix1, hv0, hv1, ov0, ov1,
          si0, si1, so0, so1):
        cid = lax.axis_index("c")
        sid = lax.axis_index("s")
        b = cid * (B // 2) + sid // 4
        q = sid % 4
        ixv = (ix0, ix1)
        hv = (hv0, hv1)
        ov = (ov0, ov1)
        sin = (si0, si1)
        sout = (so0, so1)
        io = _io16()
        io8 = io & 7
        m8 = io < 8
        pbase = [jnp.full((16,), p * CELLS * QW, jnp.int32) + io8
                 for p in range(3)]
        q8 = q * QW
        neg = jnp.full((16,), -jnp.inf, jnp.float32)

        def idx_copies(g, s, want_h):
            off = pl.multiple_of(g * K, K)
            cps = [pltpu.make_async_copy(
                cells8.at[pl.ds(b * 3 * N + p * N + off, K)],
                ixv[s].at[pl.ds(p * K, K)], sin[s])
                for p in range(3)]
            if want_h:
                hoff = pl.multiple_of(b * N * DIM + g * (K * DIM), K * DIM)
                cps.append(pltpu.make_async_copy(
                    hflat.at[pl.ds(hoff, K * DIM)], hv[s], sin[s]))
            return cps

        def issue(g, s, want_h):
            for c in idx_copies(g, s, want_h):
                c.start()

        def drain(g, s, want_h):
            for c in idx_copies(g, s, want_h):
                c.wait()

        def out_copy(g, s):
            base = (b * 4 + q) * (N * QW)
            off = pl.multiple_of(g * (K * QW), K * QW)
            return pltpu.make_async_copy(
                ov[s], pooled.at[pl.ds(base + off, K * QW)], sout[s])

        @pl.loop(0, GRIDW // 16)
        def _(i):
            grid_v[pl.ds(pl.multiple_of(i * 16, 16), 16)] = neg

        def scatter_chunk(s):
            @pl.loop(0, K // 16)
            def _(g16):
                goff = pl.multiple_of(g16 * 16, 16)
                ios = [ixv[s][pl.ds(p * K + goff, 16)] for p in range(3)]
                for j in range(16):
                    jv = jnp.full((16,), j, jnp.int32)
                    fidx = (goff + j) * DIM + q8 + io8
                    fj = plsc.load_gather(hv[s], [fidx])
                    for p in range(3):
                        off = _dg(ios[p], jv) + pbase[p]
                        g0 = plsc.load_gather(grid_v, [off], mask=m8)
                        plsc.store_scatter(grid_v, [off],
                                           jnp.maximum(g0, fj), mask=m8)

        issue(0, 0, True)
        issue(1, 1, True)

        @pl.loop(0, nch // 2)
        def _(gg):
            for s in range(2):
                g = gg * 2 + s
                drain(g, s, True)

                @pl.when(g + 2 < nch)
                def _():
                    issue(g + 2, s, True)

                scatter_chunk(s)

        issue(0, 0, False)
        issue(1, 1, False)

        @pl.loop(0, nch // 2)
        def _(gg):
            for s in range(2):
                g = gg * 2 + s
                drain(g, s, False)

                @pl.when(g + 2 < nch)
                def _():
                    issue(g + 2, s, False)

                @pl.when(g >= 2)
                def _():
                    out_copy(g - 2, s).wait()

                @pl.loop(0, K // 16)
                def _(g16):
                    goff = pl.multiple_of(g16 * 16, 16)
                    ios = [ixv[s][pl.ds(p * K + goff, 16)] for p in range(3)]
                    for j in range(16):
                        jv = jnp.full((16,), j, jnp.int32)
                        acc = plsc.load_gather(
                            grid_v, [_dg(ios[0], jv) + pbase[0]], mask=m8)
                        for p in (1, 2):
                            acc = acc + plsc.load_gather(
                                grid_v, [_dg(ios[p], jv) + pbase[p]], mask=m8)
                        plsc.store_scatter(
                            ov[s], [jnp.full((16,), (goff + j) * QW, jnp.int32)
                                    + io8],
                            acc, mask=m8)

                out_copy(g, s).start()

        out_copy(nch - 2, 0).wait()
        out_copy(nch - 1, 1).wait()

    return k


def _sc_mean_fn(B, N):
    mesh = plsc.VectorSubcoreMesh(core_axis_name="c", subcore_axis_name="s")
    nch = N // K

    @functools.partial(
        pl.kernel, mesh=mesh,
        out_type=(jax.ShapeDtypeStruct((B * 4 * GRIDW,), jnp.float32),
                  jax.ShapeDtypeStruct((B * NPL * CELLS * QW,), jnp.float32)),
        scratch_types=[
            pltpu.VMEM((GRIDW,), jnp.float32),
            pltpu.VMEM((CELLS,), jnp.float32),
            pltpu.VMEM((NPL * K,), jnp.int32), pltpu.VMEM((NPL * K,), jnp.int32),
            pltpu.VMEM((K * DIM,), jnp.float32), pltpu.VMEM((K * DIM,), jnp.float32),
            pltpu.SemaphoreType.DMA, pltpu.SemaphoreType.DMA,
        ],
        compiler_params=pltpu.CompilerParams(needs_layout_passes=False),
    )
    def k(cells8, cflat, sums, cnt, grid_v, cnt_v, ix0, ix1, hv0, hv1,
          si0, si1):
        cid = lax.axis_index("c")
        sid = lax.axis_index("s")
        b = cid * (B // 2) + sid // 4
        q = sid % 4
        ixv = (ix0, ix1)
        hv = (hv0, hv1)
        sin = (si0, si1)
        io = _io16()
        io8 = io & 7
        m8 = io < 8
        m1 = io < 1
        ones = jnp.full((16,), 1.0, jnp.float32)
        zeros = jnp.zeros((16,), jnp.float32)
        pbase = [jnp.full((16,), p * CELLS * QW, jnp.int32) + io8
                 for p in range(3)]
        q8 = q * QW

        def idx_copies(g, s):
            off = pl.multiple_of(g * K, K)
            cps = [pltpu.make_async_copy(
                cells8.at[pl.ds(b * 3 * N + p * N + off, K)],
                ixv[s].at[pl.ds(p * K, K)], sin[s])
                for p in range(3)]
            hoff = pl.multiple_of(b * N * DIM + g * (K * DIM), K * DIM)
            cps.append(pltpu.make_async_copy(
                cflat.at[pl.ds(hoff, K * DIM)], hv[s], sin[s]))
            return cps

        @pl.loop(0, GRIDW // 16)
        def _(i):
            grid_v[pl.ds(pl.multiple_of(i * 16, 16), 16)] = zeros

        @pl.loop(0, CELLS // 16)
        def _(i):
            cnt_v[pl.ds(pl.multiple_of(i * 16, 16), 16)] = zeros

        for c in idx_copies(0, 0):
            c.start()
        for c in idx_copies(1, 1):
            c.start()

        @pl.loop(0, nch // 2)
        def _(gg):
            for s in range(2):
                g = gg * 2 + s
                for c in idx_copies(g, s):
                    c.wait()

                @pl.when(g + 2 < nch)
                def _():
                    for c in idx_copies(g + 2, s):
                        c.start()

                @pl.loop(0, K // 16)
                def _(g16):
                    goff = pl.multiple_of(g16 * 16, 16)
                    ios = [ixv[s][pl.ds(p * K + goff, 16)] for p in range(3)]
                    for j in range(16):
                        jv = jnp.full((16,), j, jnp.int32)
                        fidx = (goff + j) * DIM + q8 + io8
                        fj = plsc.load_gather(hv[s], [fidx])
                        bps = [_dg(ios[p], jv) for p in range(3)]
                        for p in range(3):
                            plsc.addupdate_scatter(
                                grid_v, [bps[p] + pbase[p]], fj, mask=m8)

                        @pl.when(q < 3)
                        def _():
                            csel = jnp.where(
                                q == 0, bps[0],
                                jnp.where(q == 1, bps[1], bps[2]))
                            plsc.addupdate_scatter(
                                cnt_v, [lax.shift_right_logical(csel, 3)],
                                ones, mask=m1)

        pltpu.sync_copy(grid_v, sums.at[pl.ds((b * 4 + q) * GRIDW, GRIDW)])

        @pl.when(q < 3)
        def _():
            nstage = (K * DIM) // QW

            @pl.loop(0, CELLS // nstage)
            def _(cch):
                @pl.loop(0, (nstage * QW) // 16)
                def _(i):
                    cbase = pl.multiple_of(cch * nstage, nstage)
                    idx = (jnp.full((16,), cbase, jnp.int32)
                           + i * 2 + lax.shift_right_logical(io, 3))
                    v = plsc.load_gather(cnt_v, [idx])
                    hv0[pl.ds(pl.multiple_of(i * 16, 16), 16)] = v
                pltpu.sync_copy(
                    hv0,
                    cnt.at[pl.ds((b * NPL + q) * (CELLS * QW)
                                 + cch * (nstage * QW), nstage * QW)])

    return k


def kernel(x, params):
    B, N, _ = x.shape
    cells8 = _cells8(x)

    h = _stem0_tc(x, params)

    sc_round = _sc_round_fn(B, N)
    for blk in params['blocks'][1:]:
        pooled = sc_round(cells8, h.reshape(-1))
        h = _round_tc(h, pooled.reshape(-1, 128), blk, B, N)

    c, cflat = _fc_tc(h, params['fc_c_W'], params['fc_c_b'], B, N)

    sums, cnt8 = _sc_mean_fn(B, N)(cells8, cflat.reshape(-1))
    tri_feat = _divide_tc(sums.reshape(-1, 128), cnt8.reshape(-1, 128), B)

    return (x[..., :3], c, tri_feat)


# speedup vs baseline: 2.3016x; 2.3016x over previous
import functools

import jax
import jax.numpy as jnp
from jax import lax
from jax.experimental import pallas as pl
from jax.experimental.pallas import tpu as pltpu
from jax.experimental.pallas import tpu_sc as plsc

RES = 64
PAD = 0.1
DIM = 32
NPL = 3
CELLS = RES * RES
QW = 8
GRIDW = NPL * CELLS * QW
K = 256
NB = 1024


def _cells8(x):
    planes = [(0, 1), (0, 2), (1, 2)]
    cs = []
    for (a, b2) in planes:
        p = jnp.stack([x[..., a], x[..., b2]], axis=-1)
        p = p / (1.0 + PAD + 1e-3) + 0.5
        p = jnp.clip(p, 0.0, 1.0 - 1e-6)
        ij = jnp.clip((p * RES).astype(jnp.int32), 0, RES - 1)
        cs.append(ij[..., 0] + RES * ij[..., 1])
    c = jnp.stack(cs, axis=1)
    return (c * 8).reshape(-1)


def _pack(v, n):
    u = jnp.reshape(v, (n // 4, 4, DIM))
    return jnp.concatenate([u[:, g, :] for g in range(4)], axis=-1)


def _unpack(p, w):
    r = p.shape[0]
    m = 128 // w
    u = jnp.stack([p[:, w * g:w * g + w] for g in range(m)], axis=1)
    return jnp.reshape(u, (r * m, w))


def _stem0_body(x_ref, sw, sb, w0, b0, w1, b1, ws, bs, out_ref):
    x = x_ref[0]
    t = jnp.dot(x, sw[...], preferred_element_type=jnp.float32) + sb[...]
    net = jnp.maximum(t, 0.0)
    net = jnp.dot(net, w0[...], preferred_element_type=jnp.float32) + b0[...]
    net = jnp.maximum(net, 0.0)
    net = jnp.dot(net, w1[...], preferred_element_type=jnp.float32) + b1[...]
    sc = jnp.dot(t, ws[...], preferred_element_type=jnp.float32) + bs[...]
    out_ref[...] = _pack(sc + net, NB)


def _stem0_tc(x, params):
    B, N, _ = x.shape
    p = params
    b0 = p['blocks'][0]
    w_args = (p['stem_W'], p['stem_b'].reshape(1, -1),
              b0['fc0_W'], b0['fc0_b'].reshape(1, -1),
              b0['fc1_W'], b0['fc1_b'].reshape(1, -1),
              b0['sc_W'], b0['sc_b'].reshape(1, -1))
    w_specs = [pl.BlockSpec(w.shape, lambda bb, i: (0, 0)) for w in w_args]
    return pl.pallas_call(
        _stem0_body,
        grid=(B, N // NB),
        in_specs=[pl.BlockSpec((1, NB, 3), lambda bb, i: (bb, i, 0))] + w_specs,
        out_specs=pl.BlockSpec((NB * DIM // 128, 128),
                               lambda bb, i: (bb * (N // NB) + i, 0)),
        out_shape=jax.ShapeDtypeStruct((B * N * DIM // 128, 128), jnp.float32),
    )(x, *w_args)


def _round_body(h_ref, p0, p1, p2, p3, w0h, wp0, wp1, wp2, wp3,
                w1d, wsh, ws0, ws1, ws2, ws3, b0t, b1t, bst, out_ref):
    nb4 = NB // 4
    hp = h_ref[...]
    acc0 = jnp.dot(jnp.maximum(hp, 0.0), w0h[...],
                   preferred_element_type=jnp.float32)
    accs = jnp.dot(hp, wsh[...], preferred_element_type=jnp.float32)
    for p, wq, wsq in ((p0, wp0, ws0), (p1, wp1, ws1),
                       (p2, wp2, ws2), (p3, wp3, ws3)):
        pq = p[...]
        acc0 = acc0 + jnp.reshape(
            jnp.dot(jnp.maximum(pq, 0.0), wq[...],
                    preferred_element_type=jnp.float32), (nb4, 128))
        accs = accs + jnp.reshape(
            jnp.dot(pq, wsq[...], preferred_element_type=jnp.float32),
            (nb4, 128))
    net = jnp.maximum(acc0 + b0t[...], 0.0)
    net = jnp.dot(net, w1d[...], preferred_element_type=jnp.float32) + b1t[...]
    out_ref[...] = accs + bst[...] + net


def _bd(w, m):
    return jnp.kron(jnp.eye(m, dtype=w.dtype), w)


def _round_tc(h, pooled, blk, B, N):
    w0, w1, ws = blk['fc0_W'], blk['fc1_W'], blk['sc_W']
    w_args = tuple(
        [_bd(w0[:DIM], 4)]
        + [_bd(w0[DIM + QW * q:DIM + QW * (q + 1)], 16) for q in range(4)]
        + [_bd(w1, 4), _bd(ws[:DIM], 4)]
        + [_bd(ws[DIM + QW * q:DIM + QW * (q + 1)], 16) for q in range(4)]
        + [jnp.tile(blk['fc0_b'], 4).reshape(1, 128),
           jnp.tile(blk['fc1_b'], 4).reshape(1, 128),
           jnp.tile(blk['sc_b'], 4).reshape(1, 128)])
    w_specs = [pl.BlockSpec(w.shape, lambda bb, i: (0, 0)) for w in w_args]
    q_specs = [
        pl.BlockSpec((NB * QW // 128, 128),
                     functools.partial(
                         lambda bb, i, q: ((bb * 4 + q) * (N // NB) + i, 0),
                         q=q))
        for q in range(4)
    ]
    hspec = pl.BlockSpec((NB * DIM // 128, 128),
                         lambda bb, i: (bb * (N // NB) + i, 0))
    return pl.pallas_call(
        _round_body,
        grid=(B, N // NB),
        in_specs=[hspec] + q_specs + w_specs,
        out_specs=hspec,
        out_shape=jax.ShapeDtypeStruct((B * N * DIM // 128, 128), jnp.float32),
    )(h, pooled, pooled, pooled, pooled, *w_args)


def _fc_body(h_ref, w, b, out_ref, outf_ref):
    c = (jnp.dot(_unpack(h_ref[...], DIM), w[...],
                 preferred_element_type=jnp.float32) + b[...])
    out_ref[0] = c
    outf_ref[...] = _pack(c, NB)


def _fc_tc(h, w, b, B, N):
    w_args = (w, b.reshape(1, -1))
    w_specs = [pl.BlockSpec(a.shape, lambda bb, i: (0, 0)) for a in w_args]
    return pl.pallas_call(
        _fc_body,
        grid=(B, N // NB),
        in_specs=[pl.BlockSpec((NB * DIM // 128, 128),
                               lambda bb, i: (bb * (N // NB) + i, 0))]
        + w_specs,
        out_specs=[pl.BlockSpec((1, NB, DIM), lambda bb, i: (bb, i, 0)),
                   pl.BlockSpec((NB * DIM // 128, 128),
                                lambda bb, i: (bb * (N // NB) + i, 0))],
        out_shape=[jax.ShapeDtypeStruct((B, N, DIM), jnp.float32),
                   jax.ShapeDtypeStruct((B * N * DIM // 128, 128),
                                        jnp.float32)],
    )(h, *w_args)


def _divide_body(s0, s1, s2, s3, cnt_ref, out_ref):
    c = jnp.maximum(_unpack(cnt_ref[...], QW), 1.0)
    out_ref[0, 0] = jnp.concatenate(
        [_unpack(r[...], QW) / c for r in (s0, s1, s2, s3)], axis=-1)


_CC = 1024


def _divide_tc(sums, cnt8, B):
    nrow = _CC * QW // 128
    nch = CELLS // _CC
    q_specs = [
        pl.BlockSpec((nrow, 128),
                     functools.partial(
                         lambda bb, p, j, q:
                         (((bb * 4 + q) * NPL + p) * nch + j, 0), q=q))
        for q in range(4)
    ]
    return pl.pallas_call(
        _divide_body,
        grid=(B, NPL, nch),
        in_specs=q_specs + [pl.BlockSpec(
            (nrow, 128), lambda bb, p, j: ((bb * NPL + p) * nch + j, 0))],
        out_specs=pl.BlockSpec((1, 1, _CC, DIM),
                               lambda bb, p, j: (bb, p, j, 0)),
        out_shape=jax.ShapeDtypeStruct((B, NPL, CELLS, DIM), jnp.float32),
    )(sums, sums, sums, sums, cnt8)


def _dg(x, idx):
    return lax.gather(
        x, idx[:, None],
        lax.GatherDimensionNumbers(
            offset_dims=(), collapsed_slice_dims=(0,), start_index_map=(0,)),
        slice_sizes=(1,),
        mode=lax.GatherScatterMode.PROMISE_IN_BOUNDS)


def _io16():
    return lax.iota(jnp.int32, 16)


def _sc_round_fn(B, N):
    mesh = plsc.VectorSubcoreMesh(core_axis_name="c", subcore_axis_name="s")
    nch = N // K

    @functools.partial(
        pl.kernel, mesh=mesh,
        out_type=jax.ShapeDtypeStruct((B * 4 * N * QW,), jnp.float32),
        scratch_types=[
            pltpu.VMEM((GRIDW,), jnp.float32),
            pltpu.VMEM((NPL * K,), jnp.int32), pltpu.VMEM((NPL * K,), jnp.int32),
            pltpu.VMEM((K * DIM,), jnp.float32), pltpu.VMEM((K * DIM,), jnp.float32),
            pltpu.VMEM((K * QW,), jnp.float32), pltpu.VMEM((K * QW,), jnp.float32),
            pltpu.SemaphoreType.DMA, pltpu.SemaphoreType.DMA,
            pltpu.SemaphoreType.DMA, pltpu.SemaphoreType.DMA,
        ],
        compiler_params=pltpu.CompilerParams(needs_layout_passes=False),
    )
    def k(cells8, hflat, pooled, grid_v, ix0, ix1, hv0, hv1, ov0, ov1,
          si0, si1, so0, so1):
        cid = lax.axis_index("c")
        sid = lax.axis_index("s")
        b = cid * (B // 2) + sid // 4
        q = sid % 4
        ixv = (ix0, ix1)
        hv = (hv0, hv1)
        ov = (ov0, ov1)
        sin = (si0, si1)
        sout = (so0, so1)
        io = _io16()
        io8 = io & 7
        m8 = io < 8
        pbase = [jnp.full((16,), p * CELLS * QW, jnp.int32) + io8
                 for p in range(3)]
        q8 = q * QW
        neg = jnp.full((16,), -jnp.inf, jnp.float32)

        def idx_copies(g, s, want_h):
            off = pl.multiple_of(g * K, K)
            cps = [pltpu.make_async_copy(
                cells8.at[pl.ds(b * 3 * N + p * N + off, K)],
                ixv[s].at[pl.ds(p * K, K)], sin[s])
                for p in range(3)]
            if want_h:
                hoff = pl.multiple_of(b * N * DIM + g * (K * DIM), K * DIM)
                cps.append(pltpu.make_async_copy(
                    hflat.at[pl.ds(hoff, K * DIM)], hv[s], sin[s]))
            return cps

        def issue(g, s, want_h):
            for c in idx_copies(g, s, want_h):
                c.start()

        def drain(g, s, want_h):
            for c in idx_copies(g, s, want_h):
                c.wait()

        def out_copy(g, s):
            base = (b * 4 + q) * (N * QW)
            off = pl.multiple_of(g * (K * QW), K * QW)
            return pltpu.make_async_copy(
                ov[s], pooled.at[pl.ds(base + off, K * QW)], sout[s])

        @pl.loop(0, GRIDW // 16)
        def _(i):
            grid_v[pl.ds(pl.multiple_of(i * 16, 16), 16)] = neg

        def scatter_chunk(s):
            @pl.loop(0, K // 16)
            def _(g16):
                goff = pl.multiple_of(g16 * 16, 16)
                ios = [ixv[s][pl.ds(p * K + goff, 16)] for p in range(3)]
                for j in range(16):
                    jv = jnp.full((16,), j, jnp.int32)
                    fidx = (goff + j) * DIM + q8 + io8
                    fj = plsc.load_gather(hv[s], [fidx])
                    for p in range(3):
                        off = _dg(ios[p], jv) + pbase[p]
                        g0 = plsc.load_gather(grid_v, [off], mask=m8)
                        plsc.store_scatter(grid_v, [off],
                                           jnp.maximum(g0, fj), mask=m8)

        issue(0, 0, True)
        issue(1, 1, True)

        @pl.loop(0, nch // 2)
        def _(gg):
            for s in range(2):
                g = gg * 2 + s
                drain(g, s, True)

                @pl.when(g + 2 < nch)
                def _():
                    issue(g + 2, s, True)

                scatter_chunk(s)

        issue(0, 0, False)
        issue(1, 1, False)

        @pl.loop(0, nch // 2)
        def _(gg):
            for s in range(2):
                g = gg * 2 + s
                drain(g, s, False)

                @pl.when(g + 2 < nch)
                def _():
                    issue(g + 2, s, False)

                @pl.when(g >= 2)
                def _():
                    out_copy(g - 2, s).wait()

                @pl.loop(0, K // 16)
                def _(g16):
                    goff = pl.multiple_of(g16 * 16, 16)
                    ios = [ixv[s][pl.ds(p * K + goff, 16)] for p in range(3)]
                    for j in range(16):
                        jv = jnp.full((16,), j, jnp.int32)
                        acc = plsc.load_gather(
                            grid_v, [_dg(ios[0], jv) + pbase[0]], mask=m8)
                        for p in (1, 2):
                            acc = acc + plsc.load_gather(
                                grid_v, [_dg(ios[p], jv) + pbase[p]], mask=m8)
                        plsc.store_scatter(
                            ov[s], [jnp.full((16,), (goff + j) * QW, jnp.int32)
                                    + io8],
                            acc, mask=m8)

                out_copy(g, s).start()

        out_copy(nch - 2, 0).wait()
        out_copy(nch - 1, 1).wait()

    return k


def _sc_mean_fn(B, N):
    mesh = plsc.VectorSubcoreMesh(core_axis_name="c", subcore_axis_name="s")
    nch = N // K

    @functools.partial(
        pl.kernel, mesh=mesh,
        out_type=(jax.ShapeDtypeStruct((B * 4 * GRIDW,), jnp.float32),
                  jax.ShapeDtypeStruct((B * NPL * CELLS * QW,), jnp.float32)),
        scratch_types=[
            pltpu.VMEM((GRIDW,), jnp.float32),
            pltpu.VMEM((CELLS,), jnp.float32),
            pltpu.VMEM((NPL * K,), jnp.int32), pltpu.VMEM((NPL * K,), jnp.int32),
            pltpu.VMEM((K * DIM,), jnp.float32), pltpu.VMEM((K * DIM,), jnp.float32),
            pltpu.SemaphoreType.DMA, pltpu.SemaphoreType.DMA,
        ],
        compiler_params=pltpu.CompilerParams(needs_layout_passes=False),
    )
    def k(cells8, cflat, sums, cnt, grid_v, cnt_v, ix0, ix1, hv0, hv1,
          si0, si1):
        cid = lax.axis_index("c")
        sid = lax.axis_index("s")
        b = cid * (B // 2) + sid // 4
        q = sid % 4
        ixv = (ix0, ix1)
        hv = (hv0, hv1)
        sin = (si0, si1)
        io = _io16()
        io8 = io & 7
        m8 = io < 8
        m1 = io < 1
        ones = jnp.full((16,), 1.0, jnp.float32)
        zeros = jnp.zeros((16,), jnp.float32)
        pbase = [jnp.full((16,), p * CELLS * QW, jnp.int32) + io8
                 for p in range(3)]
        q8 = q * QW

        def idx_copies(g, s):
            off = pl.multiple_of(g * K, K)
            cps = [pltpu.make_async_copy(
                cells8.at[pl.ds(b * 3 * N + p * N + off, K)],
                ixv[s].at[pl.ds(p * K, K)], sin[s])
                for p in range(3)]
            hoff = pl.multiple_of(b * N * DIM + g * (K * DIM), K * DIM)
            cps.append(pltpu.make_async_copy(
                cflat.at[pl.ds(hoff, K * DIM)], hv[s], sin[s]))
            return cps

        @pl.loop(0, GRIDW // 16)
        def _(i):
            grid_v[pl.ds(pl.multiple_of(i * 16, 16), 16)] = zeros

        @pl.loop(0, CELLS // 16)
        def _(i):
            cnt_v[pl.ds(pl.multiple_of(i * 16, 16), 16)] = zeros

        for c in idx_copies(0, 0):
            c.start()
        for c in idx_copies(1, 1):
            c.start()

        @pl.loop(0, nch // 2)
        def _(gg):
            for s in range(2):
                g = gg * 2 + s
                for c in idx_copies(g, s):
                    c.wait()

                @pl.when(g + 2 < nch)
                def _():
                    for c in idx_copies(g + 2, s):
                        c.start()

                @pl.loop(0, K // 16)
                def _(g16):
                    goff = pl.multiple_of(g16 * 16, 16)
                    ios = [ixv[s][pl.ds(p * K + goff, 16)] for p in range(3)]
                    for j in range(16):
                        jv = jnp.full((16,), j, jnp.int32)
                        fidx = (goff + j) * DIM + q8 + io8
                        fj = plsc.load_gather(hv[s], [fidx])
                        bps = [_dg(ios[p], jv) for p in range(3)]
                        for p in range(3):
                            plsc.addupdate_scatter(
                                grid_v, [bps[p] + pbase[p]], fj, mask=m8)

                        @pl.when(q < 3)
                        def _():
                            csel = jnp.where(
                                q == 0, bps[0],
                                jnp.where(q == 1, bps[1], bps[2]))
                            plsc.addupdate_scatter(
                                cnt_v, [lax.shift_right_logical(csel, 3)],
                                ones, mask=m1)

        pltpu.sync_copy(grid_v, sums.at[pl.ds((b * 4 + q) * GRIDW, GRIDW)])

        @pl.when(q < 3)
        def _():
            nstage = (K * DIM) // QW

            @pl.loop(0, CELLS // nstage)
            def _(cch):
                @pl.loop(0, (nstage * QW) // 16)
                def _(i):
                    cbase = pl.multiple_of(cch * nstage, nstage)
                    idx = (jnp.full((16,), cbase, jnp.int32)
                           + i * 2 + lax.shift_right_logical(io, 3))
                    v = plsc.load_gather(cnt_v, [idx])
                    hv0[pl.ds(pl.multiple_of(i * 16, 16), 16)] = v
                pltpu.sync_copy(
                    hv0,
                    cnt.at[pl.ds((b * NPL + q) * (CELLS * QW)
                                 + cch * (nstage * QW), nstage * QW)])

    return k


def kernel(x, params):
    B, N, _ = x.shape
    cells8 = _cells8(x)

    h = _stem0_tc(x, params)

    sc_round = _sc_round_fn(B, N)
    for blk in params['blocks'][1:]:
        pooled = sc_round(cells8, h.reshape(-1))
        h = _round_tc(h, pooled.reshape(-1, 128), blk, B, N)

    c, cflat = _fc_tc(h, params['fc_c_W'], params['fc_c_b'], B, N)

    sums, cnt8 = _sc_mean_fn(B, N)(cells8, cflat.reshape(-1))
    tri_feat = _divide_tc(sums.reshape(-1, 128), cnt8.reshape(-1, 128), B)

    return (x[..., :3], c, tri_feat)


# speedup vs baseline: 2.3230x; 1.0093x over previous
import functools

import jax
import jax.numpy as jnp
from jax import lax
from jax.experimental import pallas as pl
from jax.experimental.pallas import tpu as pltpu
from jax.experimental.pallas import tpu_sc as plsc

RES = 64
PAD = 0.1
DIM = 32
NPL = 3
CELLS = RES * RES
QW = 8
GRIDW = NPL * CELLS * QW
K = 256
NB = 1024


def _cells8(x):
    planes = [(0, 1), (0, 2), (1, 2)]
    cs = []
    for (a, b2) in planes:
        p = jnp.stack([x[..., a], x[..., b2]], axis=-1)
        p = p / (1.0 + PAD + 1e-3) + 0.5
        p = jnp.clip(p, 0.0, 1.0 - 1e-6)
        ij = jnp.clip((p * RES).astype(jnp.int32), 0, RES - 1)
        cs.append(ij[..., 0] + RES * ij[..., 1])
    c = jnp.stack(cs, axis=1)
    return (c * 8).reshape(-1)


def _pack(v, n):
    u = jnp.reshape(v, (n // 4, 4, DIM))
    return jnp.concatenate([u[:, g, :] for g in range(4)], axis=-1)


def _unpack(p, w):
    r = p.shape[0]
    m = 128 // w
    u = jnp.stack([p[:, w * g:w * g + w] for g in range(m)], axis=1)
    return jnp.reshape(u, (r * m, w))


def _stem0_body(x_ref, sw, sb, w0, b0, w1, b1, ws, bs, out_ref):
    x = x_ref[0]
    t = jnp.dot(x, sw[...], preferred_element_type=jnp.float32) + sb[...]
    net = jnp.maximum(t, 0.0)
    net = jnp.dot(net, w0[...], preferred_element_type=jnp.float32) + b0[...]
    net = jnp.maximum(net, 0.0)
    net = jnp.dot(net, w1[...], preferred_element_type=jnp.float32) + b1[...]
    sc = jnp.dot(t, ws[...], preferred_element_type=jnp.float32) + bs[...]
    out_ref[...] = _pack(sc + net, NB)


def _stem0_tc(x, params):
    B, N, _ = x.shape
    p = params
    b0 = p['blocks'][0]
    w_args = (p['stem_W'], p['stem_b'].reshape(1, -1),
              b0['fc0_W'], b0['fc0_b'].reshape(1, -1),
              b0['fc1_W'], b0['fc1_b'].reshape(1, -1),
              b0['sc_W'], b0['sc_b'].reshape(1, -1))
    w_specs = [pl.BlockSpec(w.shape, lambda bb, i: (0, 0)) for w in w_args]
    return pl.pallas_call(
        _stem0_body,
        grid=(B, N // NB),
        in_specs=[pl.BlockSpec((1, NB, 3), lambda bb, i: (bb, i, 0))] + w_specs,
        out_specs=pl.BlockSpec((NB * DIM // 128, 128),
                               lambda bb, i: (bb * (N // NB) + i, 0)),
        out_shape=jax.ShapeDtypeStruct((B * N * DIM // 128, 128), jnp.float32),
    )(x, *w_args)


def _round_body(h_ref, p0, p1, p2, p3, w0h, wp0, wp1, wp2, wp3,
                w1d, wsh, ws0, ws1, ws2, ws3, b0t, b1t, bst, out_ref):
    nb4 = NB // 4
    hp = h_ref[...]
    acc0 = jnp.dot(jnp.maximum(hp, 0.0), w0h[...],
                   preferred_element_type=jnp.float32)
    accs = jnp.dot(hp, wsh[...], preferred_element_type=jnp.float32)
    for p, wq, wsq in ((p0, wp0, ws0), (p1, wp1, ws1),
                       (p2, wp2, ws2), (p3, wp3, ws3)):
        pq = p[...]
        acc0 = acc0 + jnp.reshape(
            jnp.dot(jnp.maximum(pq, 0.0), wq[...],
                    preferred_element_type=jnp.float32), (nb4, 128))
        accs = accs + jnp.reshape(
            jnp.dot(pq, wsq[...], preferred_element_type=jnp.float32),
            (nb4, 128))
    net = jnp.maximum(acc0 + b0t[...], 0.0)
    net = jnp.dot(net, w1d[...], preferred_element_type=jnp.float32) + b1t[...]
    out_ref[...] = accs + bst[...] + net


def _bd(w, m):
    return jnp.kron(jnp.eye(m, dtype=w.dtype), w)


def _round_tc(h, pooled, blk, B, N):
    w0, w1, ws = blk['fc0_W'], blk['fc1_W'], blk['sc_W']
    w_args = tuple(
        [_bd(w0[:DIM], 4)]
        + [_bd(w0[DIM + QW * q:DIM + QW * (q + 1)], 16) for q in range(4)]
        + [_bd(w1, 4), _bd(ws[:DIM], 4)]
        + [_bd(ws[DIM + QW * q:DIM + QW * (q + 1)], 16) for q in range(4)]
        + [jnp.tile(blk['fc0_b'], 4).reshape(1, 128),
           jnp.tile(blk['fc1_b'], 4).reshape(1, 128),
           jnp.tile(blk['sc_b'], 4).reshape(1, 128)])
    w_specs = [pl.BlockSpec(w.shape, lambda bb, i: (0, 0)) for w in w_args]
    q_specs = [
        pl.BlockSpec((NB * QW // 128, 128),
                     functools.partial(
                         lambda bb, i, q: ((bb * 4 + q) * (N // NB) + i, 0),
                         q=q))
        for q in range(4)
    ]
    hspec = pl.BlockSpec((NB * DIM // 128, 128),
                         lambda bb, i: (bb * (N // NB) + i, 0))
    return pl.pallas_call(
        _round_body,
        grid=(B, N // NB),
        in_specs=[hspec] + q_specs + w_specs,
        out_specs=hspec,
        out_shape=jax.ShapeDtypeStruct((B * N * DIM // 128, 128), jnp.float32),
    )(h, pooled, pooled, pooled, pooled, *w_args)


def _fc_body(h_ref, w, b, out_ref, outf_ref):
    c = (jnp.dot(_unpack(h_ref[...], DIM), w[...],
                 preferred_element_type=jnp.float32) + b[...])
    out_ref[0] = c
    outf_ref[...] = _pack(c, NB)


def _fc_tc(h, w, b, B, N):
    w_args = (w, b.reshape(1, -1))
    w_specs = [pl.BlockSpec(a.shape, lambda bb, i: (0, 0)) for a in w_args]
    return pl.pallas_call(
        _fc_body,
        grid=(B, N // NB),
        in_specs=[pl.BlockSpec((NB * DIM // 128, 128),
                               lambda bb, i: (bb * (N // NB) + i, 0))]
        + w_specs,
        out_specs=[pl.BlockSpec((1, NB, DIM), lambda bb, i: (bb, i, 0)),
                   pl.BlockSpec((NB * DIM // 128, 128),
                                lambda bb, i: (bb * (N // NB) + i, 0))],
        out_shape=[jax.ShapeDtypeStruct((B, N, DIM), jnp.float32),
                   jax.ShapeDtypeStruct((B * N * DIM // 128, 128),
                                        jnp.float32)],
    )(h, *w_args)


def _divide_body(s0, s1, s2, s3, cnt_ref, out_ref):
    c = jnp.maximum(_unpack(cnt_ref[...], QW), 1.0)
    out_ref[0, 0] = jnp.concatenate(
        [_unpack(r[...], QW) / c for r in (s0, s1, s2, s3)], axis=-1)


_CC = 1024


def _divide_tc(sums, cnt8, B):
    nrow = _CC * QW // 128
    nch = CELLS // _CC
    q_specs = [
        pl.BlockSpec((nrow, 128),
                     functools.partial(
                         lambda bb, p, j, q:
                         (((bb * 4 + q) * NPL + p) * nch + j, 0), q=q))
        for q in range(4)
    ]
    return pl.pallas_call(
        _divide_body,
        grid=(B, NPL, nch),
        in_specs=q_specs + [pl.BlockSpec(
            (nrow, 128), lambda bb, p, j: ((bb * NPL + p) * nch + j, 0))],
        out_specs=pl.BlockSpec((1, 1, _CC, DIM),
                               lambda bb, p, j: (bb, p, j, 0)),
        out_shape=jax.ShapeDtypeStruct((B, NPL, CELLS, DIM), jnp.float32),
    )(sums, sums, sums, sums, cnt8)


def _dg(x, idx):
    return lax.gather(
        x, idx[:, None],
        lax.GatherDimensionNumbers(
            offset_dims=(), collapsed_slice_dims=(0,), start_index_map=(0,)),
        slice_sizes=(1,),
        mode=lax.GatherScatterMode.PROMISE_IN_BOUNDS)


def _io16():
    return lax.iota(jnp.int32, 16)


def _sc_round_fn(B, N):
    mesh = plsc.VectorSubcoreMesh(core_axis_name="c", subcore_axis_name="s")
    nch = N // K

    @functools.partial(
        pl.kernel, mesh=mesh,
        out_type=jax.ShapeDtypeStruct((B * 4 * N * QW,), jnp.float32),
        scratch_types=[
            pltpu.VMEM((CELLS * QW,), jnp.float32),
            pltpu.VMEM((CELLS * QW,), jnp.float32),
            pltpu.VMEM((CELLS * QW,), jnp.float32),
            pltpu.VMEM((NPL * K,), jnp.int32), pltpu.VMEM((NPL * K,), jnp.int32),
            pltpu.VMEM((K * DIM,), jnp.float32), pltpu.VMEM((K * DIM,), jnp.float32),
            pltpu.VMEM((K * QW,), jnp.float32), pltpu.VMEM((K * QW,), jnp.float32),
            pltpu.SemaphoreType.DMA, pltpu.SemaphoreType.DMA,
            pltpu.SemaphoreType.DMA, pltpu.SemaphoreType.DMA,
        ],
        compiler_params=pltpu.CompilerParams(needs_layout_passes=False),
    )
    def k(cells8, hflat, pooled, g0_v, g1_v, g2_v, ix0, ix1, hv0, hv1,
          ov0, ov1, si0, si1, so0, so1):
        gv = (g0_v, g1_v, g2_v)
        cid = lax.axis_index("c")
        sid = lax.axis_index("s")
        b = cid * (B // 2) + sid // 4
        q = sid % 4
        ixv = (ix0, ix1)
        hv = (hv0, hv1)
        ov = (ov0, ov1)
        sin = (si0, si1)
        sout = (so0, so1)
        io = _io16()
        io8 = io & 7
        m8 = io < 8
        q8 = q * QW
        neg = jnp.full((16,), -jnp.inf, jnp.float32)

        def idx_copies(g, s, want_h):
            off = pl.multiple_of(g * K, K)
            cps = [pltpu.make_async_copy(
                cells8.at[pl.ds(b * 3 * N + p * N + off, K)],
                ixv[s].at[pl.ds(p * K, K)], sin[s])
                for p in range(3)]
            if want_h:
                hoff = pl.multiple_of(b * N * DIM + g * (K * DIM), K * DIM)
                cps.append(pltpu.make_async_copy(
                    hflat.at[pl.ds(hoff, K * DIM)], hv[s], sin[s]))
            return cps

        def issue(g, s, want_h):
            for c in idx_copies(g, s, want_h):
                c.start()

        def drain(g, s, want_h):
            for c in idx_copies(g, s, want_h):
                c.wait()

        def out_copy(g, s):
            base = (b * 4 + q) * (N * QW)
            off = pl.multiple_of(g * (K * QW), K * QW)
            return pltpu.make_async_copy(
                ov[s], pooled.at[pl.ds(base + off, K * QW)], sout[s])

        @pl.loop(0, CELLS * QW // 16)
        def _(i):
            off = pl.ds(pl.multiple_of(i * 16, 16), 16)
            g0_v[off] = neg
            g1_v[off] = neg
            g2_v[off] = neg

        def scatter_chunk(s):
            @pl.loop(0, K // 16)
            def _(g16):
                goff = pl.multiple_of(g16 * 16, 16)
                ios = [ixv[s][pl.ds(p * K + goff, 16)] for p in range(3)]
                for j in range(16):
                    jv = jnp.full((16,), j, jnp.int32)
                    fidx = (goff + j) * DIM + q8 + io8
                    fj = plsc.load_gather(hv[s], [fidx])
                    for p in range(3):
                        off = _dg(ios[p], jv) + io8
                        g0 = plsc.load_gather(gv[p], [off], mask=m8)
                        plsc.store_scatter(gv[p], [off],
                                           jnp.maximum(g0, fj), mask=m8)

        issue(0, 0, True)
        issue(1, 1, True)

        @pl.loop(0, nch // 2)
        def _(gg):
            for s in range(2):
                g = gg * 2 + s
                drain(g, s, True)

                @pl.when(g + 2 < nch)
                def _():
                    issue(g + 2, s, True)

                scatter_chunk(s)

        issue(0, 0, False)
        issue(1, 1, False)

        @pl.loop(0, nch // 2)
        def _(gg):
            for s in range(2):
                g = gg * 2 + s
                drain(g, s, False)

                @pl.when(g + 2 < nch)
                def _():
                    issue(g + 2, s, False)

                @pl.when(g >= 2)
                def _():
                    out_copy(g - 2, s).wait()

                @pl.loop(0, K // 16)
                def _(g16):
                    goff = pl.multiple_of(g16 * 16, 16)
                    ios = [ixv[s][pl.ds(p * K + goff, 16)] for p in range(3)]
                    for j in range(16):
                        jv = jnp.full((16,), j, jnp.int32)
                        acc = plsc.load_gather(
                            gv[0], [_dg(ios[0], jv) + io8], mask=m8)
                        for p in (1, 2):
                            acc = acc + plsc.load_gather(
                                gv[p], [_dg(ios[p], jv) + io8], mask=m8)
                        plsc.store_scatter(
                            ov[s], [jnp.full((16,), (goff + j) * QW, jnp.int32)
                                    + io8],
                            acc, mask=m8)

                out_copy(g, s).start()

        out_copy(nch - 2, 0).wait()
        out_copy(nch - 1, 1).wait()

    return k


def _sc_mean_fn(B, N):
    mesh = plsc.VectorSubcoreMesh(core_axis_name="c", subcore_axis_name="s")
    nch = N // K

    @functools.partial(
        pl.kernel, mesh=mesh,
        out_type=(jax.ShapeDtypeStruct((B * 4 * GRIDW,), jnp.float32),
                  jax.ShapeDtypeStruct((B * NPL * CELLS * QW,), jnp.float32)),
        scratch_types=[
            pltpu.VMEM((CELLS * QW,), jnp.float32),
            pltpu.VMEM((CELLS * QW,), jnp.float32),
            pltpu.VMEM((CELLS * QW,), jnp.float32),
            pltpu.VMEM((CELLS,), jnp.float32),
            pltpu.VMEM((NPL * K,), jnp.int32), pltpu.VMEM((NPL * K,), jnp.int32),
            pltpu.VMEM((K * DIM,), jnp.float32), pltpu.VMEM((K * DIM,), jnp.float32),
            pltpu.SemaphoreType.DMA, pltpu.SemaphoreType.DMA,
        ],
        compiler_params=pltpu.CompilerParams(needs_layout_passes=False),
    )
    def k(cells8, cflat, sums, cnt, g0_v, g1_v, g2_v, cnt_v, ix0, ix1,
          hv0, hv1, si0, si1):
        gv = (g0_v, g1_v, g2_v)
        cid = lax.axis_index("c")
        sid = lax.axis_index("s")
        b = cid * (B // 2) + sid // 4
        q = sid % 4
        ixv = (ix0, ix1)
        hv = (hv0, hv1)
        sin = (si0, si1)
        io = _io16()
        io8 = io & 7
        m8 = io < 8
        m1 = io < 1
        ones = jnp.full((16,), 1.0, jnp.float32)
        zeros = jnp.zeros((16,), jnp.float32)
        q8 = q * QW

        def idx_copies(g, s):
            off = pl.multiple_of(g * K, K)
            cps = [pltpu.make_async_copy(
                cells8.at[pl.ds(b * 3 * N + p * N + off, K)],
                ixv[s].at[pl.ds(p * K, K)], sin[s])
                for p in range(3)]
            hoff = pl.multiple_of(b * N * DIM + g * (K * DIM), K * DIM)
            cps.append(pltpu.make_async_copy(
                cflat.at[pl.ds(hoff, K * DIM)], hv[s], sin[s]))
            return cps

        @pl.loop(0, CELLS * QW // 16)
        def _(i):
            off = pl.ds(pl.multiple_of(i * 16, 16), 16)
            g0_v[off] = zeros
            g1_v[off] = zeros
            g2_v[off] = zeros

        @pl.loop(0, CELLS // 16)
        def _(i):
            cnt_v[pl.ds(pl.multiple_of(i * 16, 16), 16)] = zeros

        for c in idx_copies(0, 0):
            c.start()
        for c in idx_copies(1, 1):
            c.start()

        @pl.loop(0, nch // 2)
        def _(gg):
            for s in range(2):
                g = gg * 2 + s
                for c in idx_copies(g, s):
                    c.wait()

                @pl.when(g + 2 < nch)
                def _():
                    for c in idx_copies(g + 2, s):
                        c.start()

                @pl.loop(0, K // 16)
                def _(g16):
                    goff = pl.multiple_of(g16 * 16, 16)
                    ios = [ixv[s][pl.ds(p * K + goff, 16)] for p in range(3)]
                    for j in range(16):
                        jv = jnp.full((16,), j, jnp.int32)
                        fidx = (goff + j) * DIM + q8 + io8
                        fj = plsc.load_gather(hv[s], [fidx])
                        bps = [_dg(ios[p], jv) for p in range(3)]
                        for p in range(3):
                            plsc.addupdate_scatter(
                                gv[p], [bps[p] + io8], fj, mask=m8)

                        @pl.when(q < 3)
                        def _():
                            csel = jnp.where(
                                q == 0, bps[0],
                                jnp.where(q == 1, bps[1], bps[2]))
                            plsc.addupdate_scatter(
                                cnt_v, [lax.shift_right_logical(csel, 3)],
                                ones, mask=m1)

        for p in range(3):
            pltpu.sync_copy(
                gv[p], sums.at[pl.ds((b * 4 + q) * GRIDW + p * CELLS * QW,
                                     CELLS * QW)])

        @pl.when(q < 3)
        def _():
            nstage = (K * DIM) // QW

            @pl.loop(0, CELLS // nstage)
            def _(cch):
                @pl.loop(0, (nstage * QW) // 16)
                def _(i):
                    cbase = pl.multiple_of(cch * nstage, nstage)
                    idx = (jnp.full((16,), cbase, jnp.int32)
                           + i * 2 + lax.shift_right_logical(io, 3))
                    v = plsc.load_gather(cnt_v, [idx])
                    hv0[pl.ds(pl.multiple_of(i * 16, 16), 16)] = v
                pltpu.sync_copy(
                    hv0,
                    cnt.at[pl.ds((b * NPL + q) * (CELLS * QW)
                                 + cch * (nstage * QW), nstage * QW)])

    return k


def kernel(x, params):
    B, N, _ = x.shape
    cells8 = _cells8(x)

    h = _stem0_tc(x, params)

    sc_round = _sc_round_fn(B, N)
    for blk in params['blocks'][1:]:
        pooled = sc_round(cells8, h.reshape(-1))
        h = _round_tc(h, pooled.reshape(-1, 128), blk, B, N)

    c, cflat = _fc_tc(h, params['fc_c_W'], params['fc_c_b'], B, N)

    sums, cnt8 = _sc_mean_fn(B, N)(cells8, cflat.reshape(-1))
    tri_feat = _divide_tc(sums.reshape(-1, 128), cnt8.reshape(-1, 128), B)

    return (x[..., :3], c, tri_feat)


# speedup vs baseline: 2.6407x; 1.1367x over previous
import functools

import jax
import jax.numpy as jnp
from jax import lax
from jax.experimental import pallas as pl
from jax.experimental.pallas import tpu as pltpu
from jax.experimental.pallas import tpu_sc as plsc

RES = 64
PAD = 0.1
DIM = 32
NPL = 3
CELLS = RES * RES
QW = 8
GRIDW = NPL * CELLS * QW
K = 256
NB = 1024


def _cells8(x):
    planes = [(0, 1), (0, 2), (1, 2)]
    cs = []
    for (a, b2) in planes:
        p = jnp.stack([x[..., a], x[..., b2]], axis=-1)
        p = p / (1.0 + PAD + 1e-3) + 0.5
        p = jnp.clip(p, 0.0, 1.0 - 1e-6)
        ij = jnp.clip((p * RES).astype(jnp.int32), 0, RES - 1)
        cs.append(ij[..., 0] + RES * ij[..., 1])
    c = jnp.stack(cs, axis=1)
    return (c * 8).reshape(-1)


def _pack(v, n):
    u = jnp.reshape(v, (n // 4, 4, DIM))
    return jnp.concatenate([u[:, g, :] for g in range(4)], axis=-1)


def _unpack(p, w):
    r = p.shape[0]
    m = 128 // w
    u = jnp.stack([p[:, w * g:w * g + w] for g in range(m)], axis=1)
    return jnp.reshape(u, (r * m, w))


def _stem0_body(x_ref, sw, sb, w0, b0, w1, b1, ws, bs, out_ref):
    x = x_ref[0]
    t = jnp.dot(x, sw[...], preferred_element_type=jnp.float32) + sb[...]
    net = jnp.maximum(t, 0.0)
    net = jnp.dot(net, w0[...], preferred_element_type=jnp.float32) + b0[...]
    net = jnp.maximum(net, 0.0)
    net = jnp.dot(net, w1[...], preferred_element_type=jnp.float32) + b1[...]
    sc = jnp.dot(t, ws[...], preferred_element_type=jnp.float32) + bs[...]
    out_ref[...] = _pack(sc + net, NB)


def _stem0_tc(x, params):
    B, N, _ = x.shape
    p = params
    b0 = p['blocks'][0]
    w_args = (p['stem_W'], p['stem_b'].reshape(1, -1),
              b0['fc0_W'], b0['fc0_b'].reshape(1, -1),
              b0['fc1_W'], b0['fc1_b'].reshape(1, -1),
              b0['sc_W'], b0['sc_b'].reshape(1, -1))
    w_specs = [pl.BlockSpec(w.shape, lambda bb, i: (0, 0)) for w in w_args]
    return pl.pallas_call(
        _stem0_body,
        grid=(B, N // NB),
        in_specs=[pl.BlockSpec((1, NB, 3), lambda bb, i: (bb, i, 0))] + w_specs,
        out_specs=pl.BlockSpec((NB * DIM // 128, 128),
                               lambda bb, i: (bb * (N // NB) + i, 0)),
        out_shape=jax.ShapeDtypeStruct((B * N * DIM // 128, 128), jnp.float32),
    )(x, *w_args)


def _round_body(h_ref, p0, p1, p2, p3, w0h, wp0, wp1, wp2, wp3,
                w1d, wsh, ws0, ws1, ws2, ws3, b0t, b1t, bst, out_ref):
    nb4 = NB // 4
    hp = h_ref[...]
    acc0 = jnp.dot(jnp.maximum(hp, 0.0), w0h[...],
                   preferred_element_type=jnp.float32)
    accs = jnp.dot(hp, wsh[...], preferred_element_type=jnp.float32)
    for p, wq, wsq in ((p0, wp0, ws0), (p1, wp1, ws1),
                       (p2, wp2, ws2), (p3, wp3, ws3)):
        pq = p[...]
        acc0 = acc0 + jnp.reshape(
            jnp.dot(jnp.maximum(pq, 0.0), wq[...],
                    preferred_element_type=jnp.float32), (nb4, 128))
        accs = accs + jnp.reshape(
            jnp.dot(pq, wsq[...], preferred_element_type=jnp.float32),
            (nb4, 128))
    net = jnp.maximum(acc0 + b0t[...], 0.0)
    net = jnp.dot(net, w1d[...], preferred_element_type=jnp.float32) + b1t[...]
    out_ref[...] = accs + bst[...] + net


def _bd(w, m):
    return jnp.kron(jnp.eye(m, dtype=w.dtype), w)


def _round_tc(h, pooled, blk, B, N):
    w0, w1, ws = blk['fc0_W'], blk['fc1_W'], blk['sc_W']
    w_args = tuple(
        [_bd(w0[:DIM], 4)]
        + [_bd(w0[DIM + QW * q:DIM + QW * (q + 1)], 16) for q in range(4)]
        + [_bd(w1, 4), _bd(ws[:DIM], 4)]
        + [_bd(ws[DIM + QW * q:DIM + QW * (q + 1)], 16) for q in range(4)]
        + [jnp.tile(blk['fc0_b'], 4).reshape(1, 128),
           jnp.tile(blk['fc1_b'], 4).reshape(1, 128),
           jnp.tile(blk['sc_b'], 4).reshape(1, 128)])
    w_specs = [pl.BlockSpec(w.shape, lambda bb, i: (0, 0)) for w in w_args]
    q_specs = [
        pl.BlockSpec((NB * QW // 128, 128),
                     functools.partial(
                         lambda bb, i, q: ((bb * 4 + q) * (N // NB) + i, 0),
                         q=q))
        for q in range(4)
    ]
    hspec = pl.BlockSpec((NB * DIM // 128, 128),
                         lambda bb, i: (bb * (N // NB) + i, 0))
    return pl.pallas_call(
        _round_body,
        grid=(B, N // NB),
        in_specs=[hspec] + q_specs + w_specs,
        out_specs=hspec,
        out_shape=jax.ShapeDtypeStruct((B * N * DIM // 128, 128), jnp.float32),
    )(h, pooled, pooled, pooled, pooled, *w_args)


def _fc_body(h_ref, w, b, out_ref, outf_ref):
    c = (jnp.dot(_unpack(h_ref[...], DIM), w[...],
                 preferred_element_type=jnp.float32) + b[...])
    out_ref[0] = c
    outf_ref[...] = _pack(c, NB)


def _fc_tc(h, w, b, B, N):
    w_args = (w, b.reshape(1, -1))
    w_specs = [pl.BlockSpec(a.shape, lambda bb, i: (0, 0)) for a in w_args]
    return pl.pallas_call(
        _fc_body,
        grid=(B, N // NB),
        in_specs=[pl.BlockSpec((NB * DIM // 128, 128),
                               lambda bb, i: (bb * (N // NB) + i, 0))]
        + w_specs,
        out_specs=[pl.BlockSpec((1, NB, DIM), lambda bb, i: (bb, i, 0)),
                   pl.BlockSpec((NB * DIM // 128, 128),
                                lambda bb, i: (bb * (N // NB) + i, 0))],
        out_shape=[jax.ShapeDtypeStruct((B, N, DIM), jnp.float32),
                   jax.ShapeDtypeStruct((B * N * DIM // 128, 128),
                                        jnp.float32)],
    )(h, *w_args)


def _divide_body(s0, s1, s2, s3, cnt_ref, out_ref):
    c = jnp.maximum(_unpack(cnt_ref[...], QW), 1.0)
    out_ref[0, 0] = jnp.concatenate(
        [_unpack(r[...], QW) / c for r in (s0, s1, s2, s3)], axis=-1)


_CC = 1024


def _divide_tc(sums, cnt8, B):
    nrow = _CC * QW // 128
    nch = CELLS // _CC
    q_specs = [
        pl.BlockSpec((nrow, 128),
                     functools.partial(
                         lambda bb, p, j, q:
                         (((bb * 4 + q) * NPL + p) * nch + j, 0), q=q))
        for q in range(4)
    ]
    return pl.pallas_call(
        _divide_body,
        grid=(B, NPL, nch),
        in_specs=q_specs + [pl.BlockSpec(
            (nrow, 128), lambda bb, p, j: ((bb * NPL + p) * nch + j, 0))],
        out_specs=pl.BlockSpec((1, 1, _CC, DIM),
                               lambda bb, p, j: (bb, p, j, 0)),
        out_shape=jax.ShapeDtypeStruct((B, NPL, CELLS, DIM), jnp.float32),
    )(sums, sums, sums, sums, cnt8)


def _dg(x, idx):
    return lax.gather(
        x, idx[:, None],
        lax.GatherDimensionNumbers(
            offset_dims=(), collapsed_slice_dims=(0,), start_index_map=(0,)),
        slice_sizes=(1,),
        mode=lax.GatherScatterMode.PROMISE_IN_BOUNDS)


def _io16():
    return lax.iota(jnp.int32, 16)


def _sc_round_fn(B, N):
    mesh = plsc.VectorSubcoreMesh(core_axis_name="c", subcore_axis_name="s")
    nch = N // K

    @functools.partial(
        pl.kernel, mesh=mesh,
        out_type=jax.ShapeDtypeStruct((B * 4 * N * QW,), jnp.float32),
        scratch_types=[
            pltpu.VMEM((CELLS * QW,), jnp.float32),
            pltpu.VMEM((CELLS * QW,), jnp.float32),
            pltpu.VMEM((CELLS * QW,), jnp.float32),
            pltpu.VMEM((NPL * K,), jnp.int32), pltpu.VMEM((NPL * K,), jnp.int32),
            pltpu.VMEM((K * DIM,), jnp.float32), pltpu.VMEM((K * DIM,), jnp.float32),
            pltpu.VMEM((K * QW,), jnp.float32), pltpu.VMEM((K * QW,), jnp.float32),
            pltpu.SemaphoreType.DMA, pltpu.SemaphoreType.DMA,
            pltpu.SemaphoreType.DMA, pltpu.SemaphoreType.DMA,
        ],
        compiler_params=pltpu.CompilerParams(needs_layout_passes=False),
    )
    def k(cells8, hflat, pooled, g0_v, g1_v, g2_v, ix0, ix1, hv0, hv1,
          ov0, ov1, si0, si1, so0, so1):
        gv = (g0_v, g1_v, g2_v)
        cid = lax.axis_index("c")
        sid = lax.axis_index("s")
        b = cid * (B // 2) + sid // 4
        q = sid % 4
        ixv = (ix0, ix1)
        hv = (hv0, hv1)
        ov = (ov0, ov1)
        sin = (si0, si1)
        sout = (so0, so1)
        io = _io16()
        io8 = io & 7
        m8 = io < 8
        q8 = q * QW
        neg = jnp.full((16,), -jnp.inf, jnp.float32)

        def idx_copies(g, s, want_h):
            off = pl.multiple_of(g * K, K)
            cps = [pltpu.make_async_copy(
                cells8.at[pl.ds(b * 3 * N + p * N + off, K)],
                ixv[s].at[pl.ds(p * K, K)], sin[s])
                for p in range(3)]
            if want_h:
                hoff = pl.multiple_of(b * N * DIM + g * (K * DIM), K * DIM)
                cps.append(pltpu.make_async_copy(
                    hflat.at[pl.ds(hoff, K * DIM)], hv[s], sin[s]))
            return cps

        def issue(g, s, want_h):
            for c in idx_copies(g, s, want_h):
                c.start()

        def drain(g, s, want_h):
            for c in idx_copies(g, s, want_h):
                c.wait()

        def out_copy(g, s):
            base = (b * 4 + q) * (N * QW)
            off = pl.multiple_of(g * (K * QW), K * QW)
            return pltpu.make_async_copy(
                ov[s], pooled.at[pl.ds(base + off, K * QW)], sout[s])

        @pl.loop(0, CELLS * QW // 16)
        def _(i):
            off = pl.ds(pl.multiple_of(i * 16, 16), 16)
            g0_v[off] = neg
            g1_v[off] = neg
            g2_v[off] = neg

        def scatter_chunk(s):
            @pl.loop(0, K // 16)
            def _(g16):
                goff = pl.multiple_of(g16 * 16, 16)
                ios = [ixv[s][pl.ds(p * K + goff, 16)] for p in range(3)]
                for j in range(0, 16, 2):
                    jv0 = jnp.full((16,), j, jnp.int32)
                    jv1 = jnp.full((16,), j + 1, jnp.int32)
                    f0 = plsc.load_gather(
                        hv[s], [(goff + j) * DIM + q8 + io8])
                    f1 = plsc.load_gather(
                        hv[s], [(goff + j + 1) * DIM + q8 + io8])
                    for p in range(3):
                        off0 = _dg(ios[p], jv0) + io8
                        off1 = _dg(ios[p], jv1) + io8
                        g0 = plsc.load_gather(gv[p], [off0], mask=m8)
                        g1 = plsc.load_gather(gv[p], [off1], mask=m8)
                        f1x = jnp.where(off0 == off1,
                                        jnp.maximum(f0, f1), f1)
                        plsc.store_scatter(gv[p], [off0],
                                           jnp.maximum(g0, f0), mask=m8)
                        plsc.store_scatter(gv[p], [off1],
                                           jnp.maximum(g1, f1x), mask=m8)

        issue(0, 0, True)
        issue(1, 1, True)

        @pl.loop(0, nch // 2)
        def _(gg):
            for s in range(2):
                g = gg * 2 + s
                drain(g, s, True)

                @pl.when(g + 2 < nch)
                def _():
                    issue(g + 2, s, True)

                scatter_chunk(s)

        issue(0, 0, False)
        issue(1, 1, False)

        @pl.loop(0, nch // 2)
        def _(gg):
            for s in range(2):
                g = gg * 2 + s
                drain(g, s, False)

                @pl.when(g + 2 < nch)
                def _():
                    issue(g + 2, s, False)

                @pl.when(g >= 2)
                def _():
                    out_copy(g - 2, s).wait()

                @pl.loop(0, K // 16)
                def _(g16):
                    goff = pl.multiple_of(g16 * 16, 16)
                    ios = [ixv[s][pl.ds(p * K + goff, 16)] for p in range(3)]
                    for j in range(16):
                        jv = jnp.full((16,), j, jnp.int32)
                        acc = plsc.load_gather(
                            gv[0], [_dg(ios[0], jv) + io8], mask=m8)
                        for p in (1, 2):
                            acc = acc + plsc.load_gather(
                                gv[p], [_dg(ios[p], jv) + io8], mask=m8)
                        plsc.store_scatter(
                            ov[s], [jnp.full((16,), (goff + j) * QW, jnp.int32)
                                    + io8],
                            acc, mask=m8)

                out_copy(g, s).start()

        out_copy(nch - 2, 0).wait()
        out_copy(nch - 1, 1).wait()

    return k


def _sc_mean_fn(B, N):
    mesh = plsc.VectorSubcoreMesh(core_axis_name="c", subcore_axis_name="s")
    nch = N // K

    @functools.partial(
        pl.kernel, mesh=mesh,
        out_type=(jax.ShapeDtypeStruct((B * 4 * GRIDW,), jnp.float32),
                  jax.ShapeDtypeStruct((B * NPL * CELLS * QW,), jnp.float32)),
        scratch_types=[
            pltpu.VMEM((CELLS * QW,), jnp.float32),
            pltpu.VMEM((CELLS * QW,), jnp.float32),
            pltpu.VMEM((CELLS * QW,), jnp.float32),
            pltpu.VMEM((CELLS,), jnp.float32),
            pltpu.VMEM((NPL * K,), jnp.int32), pltpu.VMEM((NPL * K,), jnp.int32),
            pltpu.VMEM((K * DIM,), jnp.float32), pltpu.VMEM((K * DIM,), jnp.float32),
            pltpu.SemaphoreType.DMA, pltpu.SemaphoreType.DMA,
        ],
        compiler_params=pltpu.CompilerParams(needs_layout_passes=False),
    )
    def k(cells8, cflat, sums, cnt, g0_v, g1_v, g2_v, cnt_v, ix0, ix1,
          hv0, hv1, si0, si1):
        gv = (g0_v, g1_v, g2_v)
        cid = lax.axis_index("c")
        sid = lax.axis_index("s")
        b = cid * (B // 2) + sid // 4
        q = sid % 4
        ixv = (ix0, ix1)
        hv = (hv0, hv1)
        sin = (si0, si1)
        io = _io16()
        io8 = io & 7
        m8 = io < 8
        m1 = io < 1
        ones = jnp.full((16,), 1.0, jnp.float32)
        zeros = jnp.zeros((16,), jnp.float32)
        q8 = q * QW

        def idx_copies(g, s):
            off = pl.multiple_of(g * K, K)
            cps = [pltpu.make_async_copy(
                cells8.at[pl.ds(b * 3 * N + p * N + off, K)],
                ixv[s].at[pl.ds(p * K, K)], sin[s])
                for p in range(3)]
            hoff = pl.multiple_of(b * N * DIM + g * (K * DIM), K * DIM)
            cps.append(pltpu.make_async_copy(
                cflat.at[pl.ds(hoff, K * DIM)], hv[s], sin[s]))
            return cps

        @pl.loop(0, CELLS * QW // 16)
        def _(i):
            off = pl.ds(pl.multiple_of(i * 16, 16), 16)
            g0_v[off] = zeros
            g1_v[off] = zeros
            g2_v[off] = zeros

        @pl.loop(0, CELLS // 16)
        def _(i):
            cnt_v[pl.ds(pl.multiple_of(i * 16, 16), 16)] = zeros

        for c in idx_copies(0, 0):
            c.start()
        for c in idx_copies(1, 1):
            c.start()

        @pl.loop(0, nch // 2)
        def _(gg):
            for s in range(2):
                g = gg * 2 + s
                for c in idx_copies(g, s):
                    c.wait()

                @pl.when(g + 2 < nch)
                def _():
                    for c in idx_copies(g + 2, s):
                        c.start()

                @pl.loop(0, K // 16)
                def _(g16):
                    goff = pl.multiple_of(g16 * 16, 16)
                    ios = [ixv[s][pl.ds(p * K + goff, 16)] for p in range(3)]
                    for j in range(16):
                        jv = jnp.full((16,), j, jnp.int32)
                        fidx = (goff + j) * DIM + q8 + io8
                        fj = plsc.load_gather(hv[s], [fidx])
                        bps = [_dg(ios[p], jv) for p in range(3)]
                        for p in range(3):
                            plsc.addupdate_scatter(
                                gv[p], [bps[p] + io8], fj, mask=m8)

                        @pl.when(q < 3)
                        def _():
                            csel = jnp.where(
                                q == 0, bps[0],
                                jnp.where(q == 1, bps[1], bps[2]))
                            plsc.addupdate_scatter(
                                cnt_v, [lax.shift_right_logical(csel, 3)],
                                ones, mask=m1)

        for p in range(3):
            pltpu.sync_copy(
                gv[p], sums.at[pl.ds((b * 4 + q) * GRIDW + p * CELLS * QW,
                                     CELLS * QW)])

        @pl.when(q < 3)
        def _():
            nstage = (K * DIM) // QW

            @pl.loop(0, CELLS // nstage)
            def _(cch):
                @pl.loop(0, (nstage * QW) // 16)
                def _(i):
                    cbase = pl.multiple_of(cch * nstage, nstage)
                    idx = (jnp.full((16,), cbase, jnp.int32)
                           + i * 2 + lax.shift_right_logical(io, 3))
                    v = plsc.load_gather(cnt_v, [idx])
                    hv0[pl.ds(pl.multiple_of(i * 16, 16), 16)] = v
                pltpu.sync_copy(
                    hv0,
                    cnt.at[pl.ds((b * NPL + q) * (CELLS * QW)
                                 + cch * (nstage * QW), nstage * QW)])

    return k


def kernel(x, params):
    B, N, _ = x.shape
    cells8 = _cells8(x)

    h = _stem0_tc(x, params)

    sc_round = _sc_round_fn(B, N)
    for blk in params['blocks'][1:]:
        pooled = sc_round(cells8, h.reshape(-1))
        h = _round_tc(h, pooled.reshape(-1, 128), blk, B, N)

    c, cflat = _fc_tc(h, params['fc_c_W'], params['fc_c_b'], B, N)

    sums, cnt8 = _sc_mean_fn(B, N)(cells8, cflat.reshape(-1))
    tri_feat = _divide_tc(sums.reshape(-1, 128), cnt8.reshape(-1, 128), B)

    return (x[..., :3], c, tri_feat)


# speedup vs baseline: 2.9286x; 1.1090x over previous
import functools

import jax
import jax.numpy as jnp
from jax import lax
from jax.experimental import pallas as pl
from jax.experimental.pallas import tpu as pltpu
from jax.experimental.pallas import tpu_sc as plsc

RES = 64
PAD = 0.1
DIM = 32
NPL = 3
CELLS = RES * RES
QW = 8
GRIDW = NPL * CELLS * QW
K = 256
NB = 1024


def _cells8(x):
    planes = [(0, 1), (0, 2), (1, 2)]
    cs = []
    for (a, b2) in planes:
        p = jnp.stack([x[..., a], x[..., b2]], axis=-1)
        p = p / (1.0 + PAD + 1e-3) + 0.5
        p = jnp.clip(p, 0.0, 1.0 - 1e-6)
        ij = jnp.clip((p * RES).astype(jnp.int32), 0, RES - 1)
        cs.append(ij[..., 0] + RES * ij[..., 1])
    c = jnp.stack(cs, axis=1)
    return (c * 8).reshape(-1)


def _pack(v, n):
    u = jnp.reshape(v, (n // 4, 4, DIM))
    return jnp.concatenate([u[:, g, :] for g in range(4)], axis=-1)


def _unpack(p, w):
    r = p.shape[0]
    m = 128 // w
    u = jnp.stack([p[:, w * g:w * g + w] for g in range(m)], axis=1)
    return jnp.reshape(u, (r * m, w))


def _stem0_body(x_ref, sw, sb, w0, b0, w1, b1, ws, bs, out_ref):
    x = x_ref[0]
    t = jnp.dot(x, sw[...], preferred_element_type=jnp.float32) + sb[...]
    net = jnp.maximum(t, 0.0)
    net = jnp.dot(net, w0[...], preferred_element_type=jnp.float32) + b0[...]
    net = jnp.maximum(net, 0.0)
    net = jnp.dot(net, w1[...], preferred_element_type=jnp.float32) + b1[...]
    sc = jnp.dot(t, ws[...], preferred_element_type=jnp.float32) + bs[...]
    out_ref[...] = _pack(sc + net, NB)


def _stem0_tc(x, params):
    B, N, _ = x.shape
    p = params
    b0 = p['blocks'][0]
    w_args = (p['stem_W'], p['stem_b'].reshape(1, -1),
              b0['fc0_W'], b0['fc0_b'].reshape(1, -1),
              b0['fc1_W'], b0['fc1_b'].reshape(1, -1),
              b0['sc_W'], b0['sc_b'].reshape(1, -1))
    w_specs = [pl.BlockSpec(w.shape, lambda bb, i: (0, 0)) for w in w_args]
    return pl.pallas_call(
        _stem0_body,
        grid=(B, N // NB),
        in_specs=[pl.BlockSpec((1, NB, 3), lambda bb, i: (bb, i, 0))] + w_specs,
        out_specs=pl.BlockSpec((NB * DIM // 128, 128),
                               lambda bb, i: (bb * (N // NB) + i, 0)),
        out_shape=jax.ShapeDtypeStruct((B * N * DIM // 128, 128), jnp.float32),
    )(x, *w_args)


def _round_body(h_ref, p0, p1, p2, p3, w0h, wp0, wp1, wp2, wp3,
                w1d, wsh, ws0, ws1, ws2, ws3, b0t, b1t, bst, out_ref):
    nb4 = NB // 4
    hp = h_ref[...]
    acc0 = jnp.dot(jnp.maximum(hp, 0.0), w0h[...],
                   preferred_element_type=jnp.float32)
    accs = jnp.dot(hp, wsh[...], preferred_element_type=jnp.float32)
    for p, wq, wsq in ((p0, wp0, ws0), (p1, wp1, ws1),
                       (p2, wp2, ws2), (p3, wp3, ws3)):
        pq = p[...]
        acc0 = acc0 + jnp.reshape(
            jnp.dot(jnp.maximum(pq, 0.0), wq[...],
                    preferred_element_type=jnp.float32), (nb4, 128))
        accs = accs + jnp.reshape(
            jnp.dot(pq, wsq[...], preferred_element_type=jnp.float32),
            (nb4, 128))
    net = jnp.maximum(acc0 + b0t[...], 0.0)
    net = jnp.dot(net, w1d[...], preferred_element_type=jnp.float32) + b1t[...]
    out_ref[...] = accs + bst[...] + net


def _bd(w, m):
    return jnp.kron(jnp.eye(m, dtype=w.dtype), w)


def _round_tc(h, pooled, blk, B, N):
    w0, w1, ws = blk['fc0_W'], blk['fc1_W'], blk['sc_W']
    w_args = tuple(
        [_bd(w0[:DIM], 4)]
        + [_bd(w0[DIM + QW * q:DIM + QW * (q + 1)], 16) for q in range(4)]
        + [_bd(w1, 4), _bd(ws[:DIM], 4)]
        + [_bd(ws[DIM + QW * q:DIM + QW * (q + 1)], 16) for q in range(4)]
        + [jnp.tile(blk['fc0_b'], 4).reshape(1, 128),
           jnp.tile(blk['fc1_b'], 4).reshape(1, 128),
           jnp.tile(blk['sc_b'], 4).reshape(1, 128)])
    w_specs = [pl.BlockSpec(w.shape, lambda bb, i: (0, 0)) for w in w_args]
    q_specs = [
        pl.BlockSpec((NB * QW // 128, 128),
                     functools.partial(
                         lambda bb, i, q: ((bb * 4 + q) * (N // NB) + i, 0),
                         q=q))
        for q in range(4)
    ]
    hspec = pl.BlockSpec((NB * DIM // 128, 128),
                         lambda bb, i: (bb * (N // NB) + i, 0))
    return pl.pallas_call(
        _round_body,
        grid=(B, N // NB),
        in_specs=[hspec] + q_specs + w_specs,
        out_specs=hspec,
        out_shape=jax.ShapeDtypeStruct((B * N * DIM // 128, 128), jnp.float32),
    )(h, pooled, pooled, pooled, pooled, *w_args)


def _fc_body(h_ref, w, b, out_ref, outf_ref):
    c = (jnp.dot(_unpack(h_ref[...], DIM), w[...],
                 preferred_element_type=jnp.float32) + b[...])
    out_ref[0] = c
    outf_ref[...] = _pack(c, NB)


def _fc_tc(h, w, b, B, N):
    w_args = (w, b.reshape(1, -1))
    w_specs = [pl.BlockSpec(a.shape, lambda bb, i: (0, 0)) for a in w_args]
    return pl.pallas_call(
        _fc_body,
        grid=(B, N // NB),
        in_specs=[pl.BlockSpec((NB * DIM // 128, 128),
                               lambda bb, i: (bb * (N // NB) + i, 0))]
        + w_specs,
        out_specs=[pl.BlockSpec((1, NB, DIM), lambda bb, i: (bb, i, 0)),
                   pl.BlockSpec((NB * DIM // 128, 128),
                                lambda bb, i: (bb * (N // NB) + i, 0))],
        out_shape=[jax.ShapeDtypeStruct((B, N, DIM), jnp.float32),
                   jax.ShapeDtypeStruct((B * N * DIM // 128, 128),
                                        jnp.float32)],
    )(h, *w_args)


def _divide_body(s0, s1, s2, s3, cnt_ref, out_ref):
    c = jnp.maximum(_unpack(cnt_ref[...], QW), 1.0)
    out_ref[0, 0] = jnp.concatenate(
        [_unpack(r[...], QW) / c for r in (s0, s1, s2, s3)], axis=-1)


_CC = 1024


def _divide_tc(sums, cnt8, B):
    nrow = _CC * QW // 128
    nch = CELLS // _CC
    q_specs = [
        pl.BlockSpec((nrow, 128),
                     functools.partial(
                         lambda bb, p, j, q:
                         (((bb * 4 + q) * NPL + p) * nch + j, 0), q=q))
        for q in range(4)
    ]
    return pl.pallas_call(
        _divide_body,
        grid=(B, NPL, nch),
        in_specs=q_specs + [pl.BlockSpec(
            (nrow, 128), lambda bb, p, j: ((bb * NPL + p) * nch + j, 0))],
        out_specs=pl.BlockSpec((1, 1, _CC, DIM),
                               lambda bb, p, j: (bb, p, j, 0)),
        out_shape=jax.ShapeDtypeStruct((B, NPL, CELLS, DIM), jnp.float32),
    )(sums, sums, sums, sums, cnt8)


def _dg(x, idx):
    return lax.gather(
        x, idx[:, None],
        lax.GatherDimensionNumbers(
            offset_dims=(), collapsed_slice_dims=(0,), start_index_map=(0,)),
        slice_sizes=(1,),
        mode=lax.GatherScatterMode.PROMISE_IN_BOUNDS)


def _io16():
    return lax.iota(jnp.int32, 16)


def _sc_round_fn(B, N):
    mesh = plsc.VectorSubcoreMesh(core_axis_name="c", subcore_axis_name="s")
    nch = N // K

    @functools.partial(
        pl.kernel, mesh=mesh,
        out_type=jax.ShapeDtypeStruct((B * 4 * N * QW,), jnp.float32),
        scratch_types=[
            pltpu.VMEM((CELLS * QW,), jnp.float32),
            pltpu.VMEM((CELLS * QW,), jnp.float32),
            pltpu.VMEM((CELLS * QW,), jnp.float32),
            pltpu.VMEM((NPL * K,), jnp.int32), pltpu.VMEM((NPL * K,), jnp.int32),
            pltpu.VMEM((K * DIM,), jnp.float32), pltpu.VMEM((K * DIM,), jnp.float32),
            pltpu.VMEM((K * QW,), jnp.float32), pltpu.VMEM((K * QW,), jnp.float32),
            pltpu.SemaphoreType.DMA, pltpu.SemaphoreType.DMA,
            pltpu.SemaphoreType.DMA, pltpu.SemaphoreType.DMA,
        ],
        compiler_params=pltpu.CompilerParams(needs_layout_passes=False),
    )
    def k(cells8, hflat, pooled, g0_v, g1_v, g2_v, ix0, ix1, hv0, hv1,
          ov0, ov1, si0, si1, so0, so1):
        gv = (g0_v, g1_v, g2_v)
        cid = lax.axis_index("c")
        sid = lax.axis_index("s")
        b = cid * (B // 2) + sid // 4
        q = sid % 4
        ixv = (ix0, ix1)
        hv = (hv0, hv1)
        ov = (ov0, ov1)
        sin = (si0, si1)
        sout = (so0, so1)
        io = _io16()
        io8 = io & 7
        m8 = io < 8
        q8 = q * QW
        neg = jnp.full((16,), -jnp.inf, jnp.float32)

        def idx_copies(g, s, want_h):
            off = pl.multiple_of(g * K, K)
            cps = [pltpu.make_async_copy(
                cells8.at[pl.ds(b * 3 * N + p * N + off, K)],
                ixv[s].at[pl.ds(p * K, K)], sin[s])
                for p in range(3)]
            if want_h:
                hoff = pl.multiple_of(b * N * DIM + g * (K * DIM), K * DIM)
                cps.append(pltpu.make_async_copy(
                    hflat.at[pl.ds(hoff, K * DIM)], hv[s], sin[s]))
            return cps

        def issue(g, s, want_h):
            for c in idx_copies(g, s, want_h):
                c.start()

        def drain(g, s, want_h):
            for c in idx_copies(g, s, want_h):
                c.wait()

        def out_copy(g, s):
            base = (b * 4 + q) * (N * QW)
            off = pl.multiple_of(g * (K * QW), K * QW)
            return pltpu.make_async_copy(
                ov[s], pooled.at[pl.ds(base + off, K * QW)], sout[s])

        @pl.loop(0, CELLS * QW // 16)
        def _(i):
            off = pl.ds(pl.multiple_of(i * 16, 16), 16)
            g0_v[off] = neg
            g1_v[off] = neg
            g2_v[off] = neg

        def scatter_chunk(s):
            @pl.loop(0, K // 16)
            def _(g16):
                goff = pl.multiple_of(g16 * 16, 16)
                ios = [ixv[s][pl.ds(p * K + goff, 16)] for p in range(3)]
                for j in range(0, 16, 4):
                    jvs = [jnp.full((16,), j + t, jnp.int32)
                           for t in range(4)]
                    fs = [plsc.load_gather(
                        hv[s], [(goff + j + t) * DIM + q8 + io8])
                        for t in range(4)]
                    for p in range(3):
                        offs = [_dg(ios[p], jvs[t]) + io8 for t in range(4)]
                        gs = [plsc.load_gather(gv[p], [offs[t]], mask=m8)
                              for t in range(4)]
                        fx = list(fs)
                        for n in range(1, 4):
                            for m in range(n):
                                fx[n] = jnp.where(offs[m] == offs[n],
                                                  jnp.maximum(fx[m], fx[n]),
                                                  fx[n])
                        for t in range(4):
                            plsc.store_scatter(gv[p], [offs[t]],
                                               jnp.maximum(gs[t], fx[t]),
                                               mask=m8)

        issue(0, 0, True)
        issue(1, 1, True)

        @pl.loop(0, nch // 2)
        def _(gg):
            for s in range(2):
                g = gg * 2 + s
                drain(g, s, True)

                @pl.when(g + 2 < nch)
                def _():
                    issue(g + 2, s, True)

                scatter_chunk(s)

        issue(0, 0, False)
        issue(1, 1, False)

        @pl.loop(0, nch // 2)
        def _(gg):
            for s in range(2):
                g = gg * 2 + s
                drain(g, s, False)

                @pl.when(g + 2 < nch)
                def _():
                    issue(g + 2, s, False)

                @pl.when(g >= 2)
                def _():
                    out_copy(g - 2, s).wait()

                @pl.loop(0, K // 16)
                def _(g16):
                    goff = pl.multiple_of(g16 * 16, 16)
                    ios = [ixv[s][pl.ds(p * K + goff, 16)] for p in range(3)]
                    for j in range(16):
                        jv = jnp.full((16,), j, jnp.int32)
                        acc = plsc.load_gather(
                            gv[0], [_dg(ios[0], jv) + io8], mask=m8)
                        for p in (1, 2):
                            acc = acc + plsc.load_gather(
                                gv[p], [_dg(ios[p], jv) + io8], mask=m8)
                        plsc.store_scatter(
                            ov[s], [jnp.full((16,), (goff + j) * QW, jnp.int32)
                                    + io8],
                            acc, mask=m8)

                out_copy(g, s).start()

        out_copy(nch - 2, 0).wait()
        out_copy(nch - 1, 1).wait()

    return k


def _sc_mean_fn(B, N):
    mesh = plsc.VectorSubcoreMesh(core_axis_name="c", subcore_axis_name="s")
    nch = N // K

    @functools.partial(
        pl.kernel, mesh=mesh,
        out_type=(jax.ShapeDtypeStruct((B * 4 * GRIDW,), jnp.float32),
                  jax.ShapeDtypeStruct((B * NPL * CELLS * QW,), jnp.float32)),
        scratch_types=[
            pltpu.VMEM((CELLS * QW,), jnp.float32),
            pltpu.VMEM((CELLS * QW,), jnp.float32),
            pltpu.VMEM((CELLS * QW,), jnp.float32),
            pltpu.VMEM((CELLS,), jnp.float32),
            pltpu.VMEM((NPL * K,), jnp.int32), pltpu.VMEM((NPL * K,), jnp.int32),
            pltpu.VMEM((K * DIM,), jnp.float32), pltpu.VMEM((K * DIM,), jnp.float32),
            pltpu.SemaphoreType.DMA, pltpu.SemaphoreType.DMA,
        ],
        compiler_params=pltpu.CompilerParams(needs_layout_passes=False),
    )
    def k(cells8, cflat, sums, cnt, g0_v, g1_v, g2_v, cnt_v, ix0, ix1,
          hv0, hv1, si0, si1):
        gv = (g0_v, g1_v, g2_v)
        cid = lax.axis_index("c")
        sid = lax.axis_index("s")
        b = cid * (B // 2) + sid // 4
        q = sid % 4
        ixv = (ix0, ix1)
        hv = (hv0, hv1)
        sin = (si0, si1)
        io = _io16()
        io8 = io & 7
        m8 = io < 8
        m1 = io < 1
        ones = jnp.full((16,), 1.0, jnp.float32)
        zeros = jnp.zeros((16,), jnp.float32)
        q8 = q * QW

        def idx_copies(g, s):
            off = pl.multiple_of(g * K, K)
            cps = [pltpu.make_async_copy(
                cells8.at[pl.ds(b * 3 * N + p * N + off, K)],
                ixv[s].at[pl.ds(p * K, K)], sin[s])
                for p in range(3)]
            hoff = pl.multiple_of(b * N * DIM + g * (K * DIM), K * DIM)
            cps.append(pltpu.make_async_copy(
                cflat.at[pl.ds(hoff, K * DIM)], hv[s], sin[s]))
            return cps

        @pl.loop(0, CELLS * QW // 16)
        def _(i):
            off = pl.ds(pl.multiple_of(i * 16, 16), 16)
            g0_v[off] = zeros
            g1_v[off] = zeros
            g2_v[off] = zeros

        @pl.loop(0, CELLS // 16)
        def _(i):
            cnt_v[pl.ds(pl.multiple_of(i * 16, 16), 16)] = zeros

        for c in idx_copies(0, 0):
            c.start()
        for c in idx_copies(1, 1):
            c.start()

        @pl.loop(0, nch // 2)
        def _(gg):
            for s in range(2):
                g = gg * 2 + s
                for c in idx_copies(g, s):
                    c.wait()

                @pl.when(g + 2 < nch)
                def _():
                    for c in idx_copies(g + 2, s):
                        c.start()

                @pl.loop(0, K // 16)
                def _(g16):
                    goff = pl.multiple_of(g16 * 16, 16)
                    ios = [ixv[s][pl.ds(p * K + goff, 16)] for p in range(3)]
                    for j in range(16):
                        jv = jnp.full((16,), j, jnp.int32)
                        fidx = (goff + j) * DIM + q8 + io8
                        fj = plsc.load_gather(hv[s], [fidx])
                        bps = [_dg(ios[p], jv) for p in range(3)]
                        for p in range(3):
                            plsc.addupdate_scatter(
                                gv[p], [bps[p] + io8], fj, mask=m8)

                        @pl.when(q < 3)
                        def _():
                            csel = jnp.where(
                                q == 0, bps[0],
                                jnp.where(q == 1, bps[1], bps[2]))
                            plsc.addupdate_scatter(
                                cnt_v, [lax.shift_right_logical(csel, 3)],
                                ones, mask=m1)

        for p in range(3):
            pltpu.sync_copy(
                gv[p], sums.at[pl.ds((b * 4 + q) * GRIDW + p * CELLS * QW,
                                     CELLS * QW)])

        @pl.when(q < 3)
        def _():
            nstage = (K * DIM) // QW

            @pl.loop(0, CELLS // nstage)
            def _(cch):
                @pl.loop(0, (nstage * QW) // 16)
                def _(i):
                    cbase = pl.multiple_of(cch * nstage, nstage)
                    idx = (jnp.full((16,), cbase, jnp.int32)
                           + i * 2 + lax.shift_right_logical(io, 3))
                    v = plsc.load_gather(cnt_v, [idx])
                    hv0[pl.ds(pl.multiple_of(i * 16, 16), 16)] = v
                pltpu.sync_copy(
                    hv0,
                    cnt.at[pl.ds((b * NPL + q) * (CELLS * QW)
                                 + cch * (nstage * QW), nstage * QW)])

    return k


def kernel(x, params):
    B, N, _ = x.shape
    cells8 = _cells8(x)

    h = _stem0_tc(x, params)

    sc_round = _sc_round_fn(B, N)
    for blk in params['blocks'][1:]:
        pooled = sc_round(cells8, h.reshape(-1))
        h = _round_tc(h, pooled.reshape(-1, 128), blk, B, N)

    c, cflat = _fc_tc(h, params['fc_c_W'], params['fc_c_b'], B, N)

    sums, cnt8 = _sc_mean_fn(B, N)(cells8, cflat.reshape(-1))
    tri_feat = _divide_tc(sums.reshape(-1, 128), cnt8.reshape(-1, 128), B)

    return (x[..., :3], c, tri_feat)
